# baseline (device time: 73989 ns/iter reference)
import jax
import jax.numpy as jnp
from jax import lax
from jax.experimental import pallas as pl
from jax.experimental.pallas import tpu as pltpu

N_DEV = 4
B_BLK = 2
SQ = 128
D = 512
H = 8
DH = 64
ROWS = B_BLK * SQ


def kernel(x, Wq, Wo, K_ext, V_ext):
    my = lax.axis_index("i")

    def prep(t):
        t = lax.dynamic_slice_in_dim(t, my * H, H, axis=2)
        t = t.transpose(0, 2, 1, 3).reshape(N_DEV, B_BLK, H, SQ, DH)
        t = jnp.roll(t[::-1], my + 1, axis=0)
        return t.reshape(N_DEV * B_BLK * H, SQ, DH)

    K_rel = prep(K_ext)
    V_rel = prep(V_ext)
    x2d = x.reshape(ROWS, D)

    def body(x_ref, wq_ref, wo_ref, k_ref, v_ref, out_ref,
             xg_ref, part_ref, rs_ref, q_ref, attn_ref,
             ag_send, ag_recv, rs_send, rs_recv):
        my_i = lax.axis_index("i")
        right = lax.rem(my_i + 1, N_DEV)
        left = lax.rem(my_i + N_DEV - 1, N_DEV)

        barrier_sem = pltpu.get_barrier_semaphore()
        for nbr in (left, right):
            pl.semaphore_signal(
                barrier_sem, inc=1,
                device_id=(nbr,), device_id_type=pl.DeviceIdType.MESH,
            )
        pl.semaphore_wait(barrier_sem, 2)

        def compute_block(r):
            q_ref[:, :] = jnp.dot(
                xg_ref[r], wq_ref[:, :], preferred_element_type=jnp.float32
            )
            for b2 in range(B_BLK):
                for h in range(H):
                    idx = (r * B_BLK + b2) * H + h
                    qs = q_ref[b2 * SQ:(b2 + 1) * SQ, h * DH:(h + 1) * DH]
                    ks = k_ref[idx]
                    s = lax.dot_general(
                        qs, ks, (((1,), (1,)), ((), ())),
                        preferred_element_type=jnp.float32,
                    ) * 0.125
                    m = jnp.max(s, axis=-1, keepdims=True)
                    p = jnp.exp(s - m)
                    l = jnp.sum(p, axis=-1, keepdims=True)
                    a = jnp.dot(
                        p, v_ref[idx], preferred_element_type=jnp.float32
                    ) / l
                    attn_ref[b2 * SQ:(b2 + 1) * SQ, h * DH:(h + 1) * DH] = a
            part_ref[r] = jnp.dot(
                attn_ref[:, :], wo_ref[:, :], preferred_element_type=jnp.float32
            )

        xg_ref[0] = x_ref[:, :]
        compute_block(0)
        for hop in range(N_DEV - 1):
            rdma = pltpu.make_async_remote_copy(
                src_ref=xg_ref.at[hop],
                dst_ref=xg_ref.at[hop + 1],
                send_sem=ag_send.at[hop],
                recv_sem=ag_recv.at[hop],
                device_id=(right,),
                device_id_type=pl.DeviceIdType.MESH,
            )
            rdma.start()
            rdma.wait()
            compute_block(hop + 1)

        for s in range(N_DEV - 1):
            rdma = pltpu.make_async_remote_copy(
                src_ref=part_ref.at[s + 1],
                dst_ref=rs_ref.at[s],
                send_sem=rs_send.at[s],
                recv_sem=rs_recv.at[s],
                device_id=(right,),
                device_id_type=pl.DeviceIdType.MESH,
            )
            rdma.start()
            rdma.wait()
            if s < N_DEV - 2:
                part_ref[s + 2] = part_ref[s + 2] + rs_ref[s]
            else:
                out_ref[:, :] = part_ref[0] + rs_ref[s]

    out = pl.pallas_call(
        body,
        out_shape=jax.ShapeDtypeStruct((ROWS, D), jnp.float32),
        in_specs=[pl.BlockSpec(memory_space=pltpu.VMEM)] * 5,
        out_specs=pl.BlockSpec(memory_space=pltpu.VMEM),
        scratch_shapes=[
            pltpu.VMEM((N_DEV, ROWS, D), jnp.float32),
            pltpu.VMEM((N_DEV, ROWS, D), jnp.float32),
            pltpu.VMEM((N_DEV - 1, ROWS, D), jnp.float32),
            pltpu.VMEM((ROWS, D), jnp.float32),
            pltpu.VMEM((ROWS, D), jnp.float32),
            pltpu.SemaphoreType.DMA((N_DEV - 1,)),
            pltpu.SemaphoreType.DMA((N_DEV - 1,)),
            pltpu.SemaphoreType.DMA((N_DEV - 1,)),
            pltpu.SemaphoreType.DMA((N_DEV - 1,)),
        ],
        compiler_params=pltpu.CompilerParams(collective_id=0),
    )(x2d, Wq, Wo, K_rel, V_rel)

    return out.reshape(B_BLK, SQ, D)


# device time: 45640 ns/iter; 1.6211x vs baseline; 1.6211x over previous
import jax
import jax.numpy as jnp
from jax import lax
from jax.experimental import pallas as pl
from jax.experimental.pallas import tpu as pltpu

N_DEV = 4
B_BLK = 2
SQ = 128
D = 512
H = 8
DH = 64
ROWS = B_BLK * SQ


def kernel(x, Wq, Wo, K_ext, V_ext):
    my = lax.axis_index("i")

    def prep(t):
        t = lax.dynamic_slice_in_dim(t, my * H, H, axis=2)
        t = t.transpose(0, 2, 1, 3).reshape(N_DEV, B_BLK, H, SQ, DH)
        t = jnp.roll(t[::-1], my + 1, axis=0)
        return t.reshape(N_DEV * B_BLK * H, SQ, DH)

    K_rel = prep(K_ext)
    V_rel = prep(V_ext)
    x2d = x.reshape(ROWS, D)

    def body(x_ref, wq_ref, wo_ref, k_ref, v_ref, out_ref,
             xg_ref, part_ref, rs_ref, q_ref, attn_ref,
             ag_send, ag_recv, rs_send, rs_recv):
        my_i = lax.axis_index("i")
        right = lax.rem(my_i + 1, N_DEV)
        left = lax.rem(my_i + N_DEV - 1, N_DEV)

        barrier_sem = pltpu.get_barrier_semaphore()
        for nbr in (left, right):
            pl.semaphore_signal(
                barrier_sem, inc=1,
                device_id=(nbr,), device_id_type=pl.DeviceIdType.MESH,
            )
        pl.semaphore_wait(barrier_sem, 2)

        def compute_block(r):
            q_ref[:, :] = jnp.dot(
                xg_ref[r], wq_ref[:, :], preferred_element_type=jnp.float32
            )
            for b2 in range(B_BLK):
                for h in range(H):
                    idx = (r * B_BLK + b2) * H + h
                    qs = q_ref[b2 * SQ:(b2 + 1) * SQ, h * DH:(h + 1) * DH]
                    ks = k_ref[idx]
                    s = lax.dot_general(
                        qs, ks, (((1,), (1,)), ((), ())),
                        preferred_element_type=jnp.float32,
                    ) * 0.125
                    m = jnp.max(s, axis=-1, keepdims=True)
                    p = jnp.exp(s - m)
                    l = jnp.sum(p, axis=-1, keepdims=True)
                    a = jnp.dot(
                        p, v_ref[idx], preferred_element_type=jnp.float32
                    ) / l
                    attn_ref[b2 * SQ:(b2 + 1) * SQ, h * DH:(h + 1) * DH] = a
            part_ref[r] = jnp.dot(
                attn_ref[:, :], wo_ref[:, :], preferred_element_type=jnp.float32
            )

        xg_ref[0] = x_ref[:, :]

        def rs_send_block(r):
            rdma = pltpu.make_async_remote_copy(
                src_ref=part_ref.at[r],
                dst_ref=rs_ref.at[r - 1],
                send_sem=rs_send.at[r - 1],
                recv_sem=rs_recv.at[r - 1],
                device_id=(lax.rem(my_i + N_DEV - r, N_DEV),),
                device_id_type=pl.DeviceIdType.MESH,
            )
            rdma.start()
            return rdma

        rs_rdmas = []
        for hop in range(N_DEV - 1):
            ag = pltpu.make_async_remote_copy(
                src_ref=xg_ref.at[hop],
                dst_ref=xg_ref.at[hop + 1],
                send_sem=ag_send.at[hop],
                recv_sem=ag_recv.at[hop],
                device_id=(right,),
                device_id_type=pl.DeviceIdType.MESH,
            )
            ag.start()
            compute_block(hop)
            if hop > 0:
                rs_rdmas.append(rs_send_block(hop))
            ag.wait()
        compute_block(N_DEV - 1)
        rs_rdmas.append(rs_send_block(N_DEV - 1))

        acc = part_ref[0]
        for j in range(N_DEV - 1):
            recv = pltpu.make_async_remote_copy(
                src_ref=part_ref.at[j + 1],
                dst_ref=rs_ref.at[j],
                send_sem=rs_send.at[j],
                recv_sem=rs_recv.at[j],
                device_id=(left,),
                device_id_type=pl.DeviceIdType.MESH,
            )
            recv.wait_recv()
            acc = acc + rs_ref[j]
        out_ref[:, :] = acc

        for rdma in rs_rdmas:
            rdma.wait_send()

    out = pl.pallas_call(
        body,
        out_shape=jax.ShapeDtypeStruct((ROWS, D), jnp.float32),
        in_specs=[pl.BlockSpec(memory_space=pltpu.VMEM)] * 5,
        out_specs=pl.BlockSpec(memory_space=pltpu.VMEM),
        scratch_shapes=[
            pltpu.VMEM((N_DEV, ROWS, D), jnp.float32),
            pltpu.VMEM((N_DEV, ROWS, D), jnp.float32),
            pltpu.VMEM((N_DEV - 1, ROWS, D), jnp.float32),
            pltpu.VMEM((ROWS, D), jnp.float32),
            pltpu.VMEM((ROWS, D), jnp.float32),
            pltpu.SemaphoreType.DMA((N_DEV - 1,)),
            pltpu.SemaphoreType.DMA((N_DEV - 1,)),
            pltpu.SemaphoreType.DMA((N_DEV - 1,)),
            pltpu.SemaphoreType.DMA((N_DEV - 1,)),
        ],
        compiler_params=pltpu.CompilerParams(collective_id=0),
    )(x2d, Wq, Wo, K_rel, V_rel)

    return out.reshape(B_BLK, SQ, D)


# device time: 39245 ns/iter; 1.8853x vs baseline; 1.1630x over previous
import jax
import jax.numpy as jnp
from jax import lax
from jax.experimental import pallas as pl
from jax.experimental.pallas import tpu as pltpu

N_DEV = 4
B_BLK = 2
SQ = 128
D = 512
H = 8
DH = 64
ROWS = B_BLK * SQ


def kernel(x, Wq, Wo, K_ext, V_ext):
    my = lax.axis_index("i")

    def prep(t):
        t = lax.dynamic_slice_in_dim(t, my * H, H, axis=2)
        t = t.transpose(0, 2, 1, 3).reshape(N_DEV, B_BLK, H, SQ, DH)
        t = jnp.roll(t[::-1], my + 1, axis=0)
        return t.reshape(N_DEV * B_BLK * H, SQ, DH)

    K_rel = prep(K_ext).astype(jnp.bfloat16)
    V_rel = prep(V_ext).astype(jnp.bfloat16)
    x2d = x.reshape(ROWS, D).astype(jnp.bfloat16)
    Wq = Wq.astype(jnp.bfloat16)
    Wo = Wo.astype(jnp.bfloat16)

    def body(x_ref, wq_ref, wo_ref, k_ref, v_ref, out_ref,
             xg_ref, part_ref, rs_ref, q_ref, attn_ref,
             ag_send, ag_recv, rs_send, rs_recv):
        my_i = lax.axis_index("i")
        right = lax.rem(my_i + 1, N_DEV)
        left = lax.rem(my_i + N_DEV - 1, N_DEV)

        barrier_sem = pltpu.get_barrier_semaphore()
        for nbr in (left, right):
            pl.semaphore_signal(
                barrier_sem, inc=1,
                device_id=(nbr,), device_id_type=pl.DeviceIdType.MESH,
            )
        pl.semaphore_wait(barrier_sem, 2)

        def compute_block(r):
            q_ref[:, :] = jnp.dot(
                xg_ref[r], wq_ref[:, :], preferred_element_type=jnp.float32
            ).astype(jnp.bfloat16)
            for b2 in range(B_BLK):
                for h in range(H):
                    idx = (r * B_BLK + b2) * H + h
                    qs = q_ref[b2 * SQ:(b2 + 1) * SQ, h * DH:(h + 1) * DH]
                    ks = k_ref[idx]
                    s = lax.dot_general(
                        qs, ks, (((1,), (1,)), ((), ())),
                        preferred_element_type=jnp.float32,
                    ) * 0.125
                    m = jnp.max(s, axis=-1, keepdims=True)
                    p = jnp.exp(s - m)
                    l = jnp.sum(p, axis=-1, keepdims=True)
                    a = jnp.dot(
                        p.astype(jnp.bfloat16), v_ref[idx],
                        preferred_element_type=jnp.float32,
                    ) / l
                    attn_ref[b2 * SQ:(b2 + 1) * SQ, h * DH:(h + 1) * DH] = (
                        a.astype(jnp.bfloat16)
                    )
            part_ref[r] = jnp.dot(
                attn_ref[:, :], wo_ref[:, :], preferred_element_type=jnp.float32
            )

        xg_ref[0] = x_ref[:, :]

        def rs_send_block(r):
            rdma = pltpu.make_async_remote_copy(
                src_ref=part_ref.at[r],
                dst_ref=rs_ref.at[r - 1],
                send_sem=rs_send.at[r - 1],
                recv_sem=rs_recv.at[r - 1],
                device_id=(lax.rem(my_i + N_DEV - r, N_DEV),),
                device_id_type=pl.DeviceIdType.MESH,
            )
            rdma.start()
            return rdma

        rs_rdmas = []
        for hop in range(N_DEV - 1):
            ag = pltpu.make_async_remote_copy(
                src_ref=xg_ref.at[hop],
                dst_ref=xg_ref.at[hop + 1],
                send_sem=ag_send.at[hop],
                recv_sem=ag_recv.at[hop],
                device_id=(right,),
                device_id_type=pl.DeviceIdType.MESH,
            )
            ag.start()
            compute_block(hop)
            if hop > 0:
                rs_rdmas.append(rs_send_block(hop))
            ag.wait()
        compute_block(N_DEV - 1)
        rs_rdmas.append(rs_send_block(N_DEV - 1))

        acc = part_ref[0]
        for j in range(N_DEV - 1):
            recv = pltpu.make_async_remote_copy(
                src_ref=part_ref.at[j + 1],
                dst_ref=rs_ref.at[j],
                send_sem=rs_send.at[j],
                recv_sem=rs_recv.at[j],
                device_id=(left,),
                device_id_type=pl.DeviceIdType.MESH,
            )
            recv.wait_recv()
            acc = acc + rs_ref[j]
        out_ref[:, :] = acc

        for rdma in rs_rdmas:
            rdma.wait_send()

    out = pl.pallas_call(
        body,
        out_shape=jax.ShapeDtypeStruct((ROWS, D), jnp.float32),
        in_specs=[pl.BlockSpec(memory_space=pltpu.VMEM)] * 5,
        out_specs=pl.BlockSpec(memory_space=pltpu.VMEM),
        scratch_shapes=[
            pltpu.VMEM((N_DEV, ROWS, D), jnp.bfloat16),
            pltpu.VMEM((N_DEV, ROWS, D), jnp.float32),
            pltpu.VMEM((N_DEV - 1, ROWS, D), jnp.float32),
            pltpu.VMEM((ROWS, D), jnp.bfloat16),
            pltpu.VMEM((ROWS, D), jnp.bfloat16),
            pltpu.SemaphoreType.DMA((N_DEV - 1,)),
            pltpu.SemaphoreType.DMA((N_DEV - 1,)),
            pltpu.SemaphoreType.DMA((N_DEV - 1,)),
            pltpu.SemaphoreType.DMA((N_DEV - 1,)),
        ],
        compiler_params=pltpu.CompilerParams(collective_id=0),
    )(x2d, Wq, Wo, K_rel, V_rel)

    return out.reshape(B_BLK, SQ, D)


# device time: 32050 ns/iter; 2.3085x vs baseline; 1.2245x over previous
import os

import jax
import jax.numpy as jnp
from jax import lax
from jax.experimental import pallas as pl
from jax.experimental.pallas import tpu as pltpu

_VARIANT = os.environ.get("KVAR", "full")

N_DEV = 4
B_BLK = 2
SQ = 128
D = 512
H = 8
DH = 64
ROWS = B_BLK * SQ


def kernel(x, Wq, Wo, K_ext, V_ext):
    my = lax.axis_index("i")

    def prep(t):
        t = lax.dynamic_slice_in_dim(t, my * H, H, axis=2)
        t = t.transpose(0, 2, 1, 3)
        return t.reshape(N_DEV * B_BLK * H, SQ, DH).astype(jnp.bfloat16)

    K_rel = prep(K_ext)
    V_rel = prep(V_ext)
    x2d = x.reshape(ROWS, D)

    def body(x_ref, wq_ref, wo_ref, k_ref, v_ref, out_ref,
             xg_ref, part_ref, rs_ref, q_ref, attn_ref, wq_b, wo_b,
             ag_send, ag_recv, rs_send, rs_recv):
        my_i = lax.axis_index("i")
        right = lax.rem(my_i + 1, N_DEV)
        left = lax.rem(my_i + N_DEV - 1, N_DEV)

        barrier_sem = pltpu.get_barrier_semaphore()
        for nbr in (left, right):
            pl.semaphore_signal(
                barrier_sem, inc=1,
                device_id=(nbr,), device_id_type=pl.DeviceIdType.MESH,
            )
        pl.semaphore_wait(barrier_sem, 2)

        def compute_block(r):
            if _VARIANT == "nocompute":
                part_ref[r] = xg_ref[r]
                return
            origin = lax.rem(my_i + N_DEV - r, N_DEV)
            kv_base = origin * (B_BLK * H)
            q_ref[:, :] = jnp.dot(
                xg_ref[r], wq_b[:, :], preferred_element_type=jnp.float32
            ).astype(jnp.bfloat16)
            if _VARIANT == "noattn":
                part_ref[r] = jnp.dot(
                    q_ref[:, :], wo_b[:, :],
                    preferred_element_type=jnp.float32,
                ).astype(jnp.bfloat16)
                return
            for b2 in range(B_BLK):
                for h in range(H):
                    idx = kv_base + b2 * H + h
                    qs = q_ref[b2 * SQ:(b2 + 1) * SQ, h * DH:(h + 1) * DH]
                    ks = k_ref[idx]
                    s = lax.dot_general(
                        qs, ks, (((1,), (1,)), ((), ())),
                        preferred_element_type=jnp.float32,
                    ) * 0.125
                    m = jnp.max(s, axis=-1, keepdims=True)
                    p = jnp.exp(s - m)
                    l = jnp.sum(p, axis=-1, keepdims=True)
                    a = jnp.dot(
                        p.astype(jnp.bfloat16), v_ref[idx],
                        preferred_element_type=jnp.float32,
                    ) / l
                    attn_ref[b2 * SQ:(b2 + 1) * SQ, h * DH:(h + 1) * DH] = (
                        a.astype(jnp.bfloat16)
                    )
            part_ref[r] = jnp.dot(
                attn_ref[:, :], wo_b[:, :], preferred_element_type=jnp.float32
            ).astype(jnp.bfloat16)

        wq_b[:, :] = wq_ref[:, :].astype(jnp.bfloat16)
        wo_b[:, :] = wo_ref[:, :].astype(jnp.bfloat16)

        if _VARIANT == "nocomm":
            xg_ref[0] = x_ref[:, :].astype(jnp.bfloat16)
            for r in range(N_DEV):
                compute_block(r)
            out_ref[:, :] = (
                part_ref[0].astype(jnp.float32)
                + part_ref[1].astype(jnp.float32)
                + part_ref[2].astype(jnp.float32)
                + part_ref[3].astype(jnp.float32)
            )
            return

        xg_ref[0] = x_ref[:, :].astype(jnp.bfloat16)

        def rs_send_block(r):
            rdma = pltpu.make_async_remote_copy(
                src_ref=part_ref.at[r],
                dst_ref=rs_ref.at[r - 1],
                send_sem=rs_send.at[r - 1],
                recv_sem=rs_recv.at[r - 1],
                device_id=(lax.rem(my_i + N_DEV - r, N_DEV),),
                device_id_type=pl.DeviceIdType.MESH,
            )
            rdma.start()
            return rdma

        rs_rdmas = []
        for hop in range(N_DEV - 1):
            ag = pltpu.make_async_remote_copy(
                src_ref=xg_ref.at[hop],
                dst_ref=xg_ref.at[hop + 1],
                send_sem=ag_send.at[hop],
                recv_sem=ag_recv.at[hop],
                device_id=(right,),
                device_id_type=pl.DeviceIdType.MESH,
            )
            ag.start()
            compute_block(hop)
            if hop > 0:
                rs_rdmas.append(rs_send_block(hop))
            ag.wait()
        compute_block(N_DEV - 1)
        rs_rdmas.append(rs_send_block(N_DEV - 1))

        def recv_desc(j):
            return pltpu.make_async_remote_copy(
                src_ref=part_ref.at[j + 1],
                dst_ref=rs_ref.at[j],
                send_sem=rs_send.at[j],
                recv_sem=rs_recv.at[j],
                device_id=(left,),
                device_id_type=pl.DeviceIdType.MESH,
            )

        recv_desc(0).wait_recv()
        recv_desc(1).wait_recv()
        acc = (
            part_ref[0].astype(jnp.float32)
            + rs_ref[0].astype(jnp.float32)
            + rs_ref[1].astype(jnp.float32)
        )
        recv_desc(2).wait_recv()
        out_ref[:, :] = acc + rs_ref[2].astype(jnp.float32)

        for rdma in rs_rdmas:
            rdma.wait_send()

    out = pl.pallas_call(
        body,
        out_shape=jax.ShapeDtypeStruct((ROWS, D), jnp.float32),
        in_specs=[pl.BlockSpec(memory_space=pltpu.VMEM)] * 5,
        out_specs=pl.BlockSpec(memory_space=pltpu.VMEM),
        scratch_shapes=[
            pltpu.VMEM((N_DEV, ROWS, D), jnp.bfloat16),
            pltpu.VMEM((N_DEV, ROWS, D), jnp.bfloat16),
            pltpu.VMEM((N_DEV - 1, ROWS, D), jnp.bfloat16),
            pltpu.VMEM((ROWS, D), jnp.bfloat16),
            pltpu.VMEM((ROWS, D), jnp.bfloat16),
            pltpu.VMEM((D, D), jnp.bfloat16),
            pltpu.VMEM((D, D), jnp.bfloat16),
            pltpu.SemaphoreType.DMA((N_DEV - 1,)),
            pltpu.SemaphoreType.DMA((N_DEV - 1,)),
            pltpu.SemaphoreType.DMA((N_DEV - 1,)),
            pltpu.SemaphoreType.DMA((N_DEV - 1,)),
        ],
        compiler_params=pltpu.CompilerParams(collective_id=0),
    )(x2d, Wq, Wo, K_rel, V_rel)

    return out.reshape(B_BLK, SQ, D)


# device time: 30867 ns/iter; 2.3970x vs baseline; 1.0383x over previous
import os

import jax
import jax.numpy as jnp
from jax import lax
from jax.experimental import pallas as pl
from jax.experimental.pallas import tpu as pltpu

_VARIANT = os.environ.get("KVAR", "full")

N_DEV = 4
B_BLK = 2
SQ = 128
D = 512
H = 8
DH = 64
ROWS = B_BLK * SQ


def kernel(x, Wq, Wo, K_ext, V_ext):
    my = lax.axis_index("i")

    def prep(t):
        t = t.reshape(N_DEV * B_BLK, SQ, 4 * H * DH)
        t = lax.dynamic_slice_in_dim(t, my * (H * DH), H * DH, axis=2)
        return t.astype(jnp.bfloat16)

    K_rel = prep(K_ext)
    V_rel = prep(V_ext)
    x2d = x.reshape(ROWS, D)

    def body(x_ref, wq_ref, wo_ref, k_ref, v_ref, out_ref,
             xg_ref, part_ref, rs_ref, q_ref, attn_ref, wq_b, wo_b,
             ag_send, ag_recv, rs_send, rs_recv):
        my_i = lax.axis_index("i")
        right = lax.rem(my_i + 1, N_DEV)
        left = lax.rem(my_i + N_DEV - 1, N_DEV)

        barrier_sem = pltpu.get_barrier_semaphore()
        for nbr in (left, right):
            pl.semaphore_signal(
                barrier_sem, inc=1,
                device_id=(nbr,), device_id_type=pl.DeviceIdType.MESH,
            )
        pl.semaphore_wait(barrier_sem, 2)

        def compute_block(r):
            if _VARIANT == "nocompute":
                part_ref[r] = xg_ref[r]
                return
            origin = lax.rem(my_i + N_DEV - r, N_DEV)
            q_ref[:, :] = jnp.dot(
                xg_ref[r], wq_b[:, :], preferred_element_type=jnp.float32
            ).astype(jnp.bfloat16)
            if _VARIANT == "noattn":
                part_ref[r] = jnp.dot(
                    q_ref[:, :], wo_b[:, :],
                    preferred_element_type=jnp.float32,
                ).astype(jnp.bfloat16)
                return
            for b2 in range(B_BLK):
                bg = origin * B_BLK + b2
                for h in range(H):
                    qs = q_ref[b2 * SQ:(b2 + 1) * SQ, h * DH:(h + 1) * DH]
                    ks = k_ref[bg, :, h * DH:(h + 1) * DH]
                    s = lax.dot_general(
                        qs, ks, (((1,), (1,)), ((), ())),
                        preferred_element_type=jnp.float32,
                    ) * 0.125
                    m = jnp.max(s, axis=-1, keepdims=True)
                    p = jnp.exp(s - m)
                    l = jnp.sum(p, axis=-1, keepdims=True)
                    a = jnp.dot(
                        p.astype(jnp.bfloat16),
                        v_ref[bg, :, h * DH:(h + 1) * DH],
                        preferred_element_type=jnp.float32,
                    ) / l
                    attn_ref[b2 * SQ:(b2 + 1) * SQ, h * DH:(h + 1) * DH] = (
                        a.astype(jnp.bfloat16)
                    )
            part_ref[r] = jnp.dot(
                attn_ref[:, :], wo_b[:, :], preferred_element_type=jnp.float32
            ).astype(jnp.bfloat16)

        wq_b[:, :] = wq_ref[:, :].astype(jnp.bfloat16)
        wo_b[:, :] = wo_ref[:, :].astype(jnp.bfloat16)

        if _VARIANT == "nocomm":
            xg_ref[0] = x_ref[:, :].astype(jnp.bfloat16)
            for r in range(N_DEV):
                compute_block(r)
            out_ref[:, :] = (
                part_ref[0].astype(jnp.float32)
                + part_ref[1].astype(jnp.float32)
                + part_ref[2].astype(jnp.float32)
                + part_ref[3].astype(jnp.float32)
            )
            return

        xg_ref[0] = x_ref[:, :].astype(jnp.bfloat16)

        def rs_send_block(r):
            rdma = pltpu.make_async_remote_copy(
                src_ref=part_ref.at[r],
                dst_ref=rs_ref.at[r - 1],
                send_sem=rs_send.at[r - 1],
                recv_sem=rs_recv.at[r - 1],
                device_id=(lax.rem(my_i + N_DEV - r, N_DEV),),
                device_id_type=pl.DeviceIdType.MESH,
            )
            rdma.start()
            return rdma

        rs_rdmas = []
        for hop in range(N_DEV - 1):
            ag = pltpu.make_async_remote_copy(
                src_ref=xg_ref.at[hop],
                dst_ref=xg_ref.at[hop + 1],
                send_sem=ag_send.at[hop],
                recv_sem=ag_recv.at[hop],
                device_id=(right,),
                device_id_type=pl.DeviceIdType.MESH,
            )
            ag.start()
            compute_block(hop)
            if hop > 0:
                rs_rdmas.append(rs_send_block(hop))
            ag.wait()
        compute_block(N_DEV - 1)
        rs_rdmas.append(rs_send_block(N_DEV - 1))

        def recv_desc(j):
            return pltpu.make_async_remote_copy(
                src_ref=part_ref.at[j + 1],
                dst_ref=rs_ref.at[j],
                send_sem=rs_send.at[j],
                recv_sem=rs_recv.at[j],
                device_id=(left,),
                device_id_type=pl.DeviceIdType.MESH,
            )

        recv_desc(0).wait_recv()
        recv_desc(1).wait_recv()
        acc = (
            part_ref[0].astype(jnp.float32)
            + rs_ref[0].astype(jnp.float32)
            + rs_ref[1].astype(jnp.float32)
        )
        recv_desc(2).wait_recv()
        out_ref[:, :] = acc + rs_ref[2].astype(jnp.float32)

        for rdma in rs_rdmas:
            rdma.wait_send()

    out = pl.pallas_call(
        body,
        out_shape=jax.ShapeDtypeStruct((ROWS, D), jnp.float32),
        in_specs=[pl.BlockSpec(memory_space=pltpu.VMEM)] * 5,
        out_specs=pl.BlockSpec(memory_space=pltpu.VMEM),
        scratch_shapes=[
            pltpu.VMEM((N_DEV, ROWS, D), jnp.bfloat16),
            pltpu.VMEM((N_DEV, ROWS, D), jnp.bfloat16),
            pltpu.VMEM((N_DEV - 1, ROWS, D), jnp.bfloat16),
            pltpu.VMEM((ROWS, D), jnp.bfloat16),
            pltpu.VMEM((ROWS, D), jnp.bfloat16),
            pltpu.VMEM((D, D), jnp.bfloat16),
            pltpu.VMEM((D, D), jnp.bfloat16),
            pltpu.SemaphoreType.DMA((N_DEV - 1,)),
            pltpu.SemaphoreType.DMA((N_DEV - 1,)),
            pltpu.SemaphoreType.DMA((N_DEV - 1,)),
            pltpu.SemaphoreType.DMA((N_DEV - 1,)),
        ],
        compiler_params=pltpu.CompilerParams(collective_id=0),
    )(x2d, Wq, Wo, K_rel, V_rel)

    return out.reshape(B_BLK, SQ, D)
